# baseline (speedup 1.0000x reference)
import jax
import jax.numpy as jnp
from jax import lax
from jax.experimental import pallas as pl
from jax.experimental.pallas import tpu as pltpu
from jax.experimental.pallas import tpu_sc as plsc

N_NODES = 10000
N_EDGES = 320000
D = 128
NP = 10240
NC, NS = 2, 16
NW = NC * NS
B = 128
CPT = 80
ER = NW * CPT
PAD_IDX = 10200
RPS = NP // NS
NBUF = 2
K = 8
CPT1 = ER // NS
NBLK = CPT1 // K
RB = 1024
GRID = NP // RB

_mesh = plsc.VectorSubcoreMesh(
    core_axis_name="c", subcore_axis_name="s", num_cores=NC, num_subcores=NS
)


def _deg_body(dst_hbm, deg_out, dstv, ones_v, zrow_v, acc, sem):
    cid = lax.axis_index("c")
    sid = lax.axis_index("s")
    wid = sid * NC + cid
    one16 = jnp.full((16,), 1.0, jnp.float32)
    zero16 = jnp.zeros((16,), jnp.float32)
    for i in range(B // 16):
        ones_v[pl.ds(i * 16, 16)] = one16
    for i in range(RPS // 16):
        zrow_v[pl.ds(i * 16, 16)] = zero16
    ro = pl.multiple_of(sid * RPS, 8)
    pltpu.sync_copy(zrow_v, acc.at[pl.ds(ro, RPS)])
    pltpu.sync_copy(dst_hbm.at[pl.ds(wid * CPT, CPT)], dstv)
    plsc.subcore_barrier()
    for g in range(CPT // 16):
        descs = [
            pltpu.async_copy(ones_v, acc.at[dstv.at[g * 16 + i]], sem, add=True)
            for i in range(16)
        ]
        for d in descs:
            d.wait()
    plsc.subcore_barrier()
    wo = pl.multiple_of(cid * NP + sid * RPS, 8)
    pltpu.sync_copy(acc.at[pl.ds(ro, RPS)], deg_out.at[pl.ds(wo, RPS)])


_deg_call = pl.kernel(
    _deg_body,
    out_type=jax.ShapeDtypeStruct((NC * NP,), jnp.float32),
    mesh=_mesh,
    scratch_types=[
        pltpu.VMEM((CPT, B), jnp.int32),
        pltpu.VMEM((B,), jnp.float32),
        pltpu.VMEM((RPS,), jnp.float32),
        pltpu.VMEM_SHARED((NP,), jnp.float32),
        pltpu.SemaphoreType.DMA,
    ],
)


def _agg_body(src_hbm, dst_hbm, g_hbm, out_hbm,
              si0, si1, si2, di0, di1, di2, r0, r1, acc,
              isem, gsem0, gsem1, ssem0, ssem1):
    cid = lax.axis_index("c")
    sid = lax.axis_index("s")
    sib = (si0, si1, si2)
    dib = (di0, di1, di2)
    bufs = (r0, r1)
    gsems = (gsem0, gsem1)
    ro = pl.multiple_of(sid * RPS, 8)
    pltpu.sync_copy(g_hbm.at[pl.ds(ro, RPS)], acc.at[pl.ds(ro, RPS)])
    plsc.subcore_barrier()
    base = sid * CPT1

    def fire_idx(kb):
        p = kb % 3
        return (
            pltpu.async_copy(src_hbm.at[pl.ds(base + kb * K, K)], sib[p], isem),
            pltpu.async_copy(dst_hbm.at[pl.ds(base + kb * K, K)], dib[p], isem),
        )

    def fire_gather(c):
        b = c % NBUF
        return pltpu.async_copy(
            g_hbm.at[sib[(c // K) % 3].at[c % K]], bufs[b], gsems[b]
        )

    @pl.when(cid == 1)
    def _edges():
        pltpu.sync_copy(src_hbm.at[pl.ds(base, K)], si0)
        pltpu.sync_copy(dst_hbm.at[pl.ds(base, K)], di0)
        idescs = {1: fire_idx(1)}
        gd = [None] * CPT1
        gd[0] = fire_gather(0)
        gd[1] = fire_gather(1)
        for c in range(CPT1):
            kb, rc = divmod(c, K)
            if rc == 1 and kb + 2 < NBLK:
                idescs[kb + 2] = fire_idx(kb + 2)
            b = c % NBUF
            gd[c].wait()
            nc = c + NBUF
            if nc < CPT1 and nc % K == 0:
                for d in idescs[nc // K]:
                    d.wait()
            pltpu.sync_copy(bufs[b], acc.at[dib[kb % 3].at[rc]], add=True)
            if nc < CPT1:
                gd[nc] = fire_gather(nc)

    plsc.subcore_barrier()
    wo = pl.multiple_of(cid * NP + sid * RPS, 8)
    pltpu.sync_copy(acc.at[pl.ds(ro, RPS)], out_hbm.at[pl.ds(wo, RPS)])


_agg_call = pl.kernel(
    _agg_body,
    out_type=jax.ShapeDtypeStruct((NC * NP, D), jnp.float32),
    mesh=_mesh,
    scratch_types=[
        pltpu.VMEM((K, B), jnp.int32),
        pltpu.VMEM((K, B), jnp.int32),
        pltpu.VMEM((K, B), jnp.int32),
        pltpu.VMEM((K, B), jnp.int32),
        pltpu.VMEM((K, B), jnp.int32),
        pltpu.VMEM((K, B), jnp.int32),
        pltpu.VMEM((B, D), jnp.float32),
        pltpu.VMEM((B, D), jnp.float32),
        pltpu.VMEM_SHARED((NP, D), jnp.float32),
        pltpu.SemaphoreType.DMA,
        pltpu.SemaphoreType.DMA,
        pltpu.SemaphoreType.DMA,
        pltpu.SemaphoreType.DMA,
        pltpu.SemaphoreType.DMA,
    ],
)


def _tc1_body(x_ref, w_ref, d0_ref, d1_ref, o_ref):
    dinv = lax.rsqrt(d0_ref[0] + d1_ref[0] + 1.0)
    h = jnp.dot(x_ref[...], w_ref[...], preferred_element_type=jnp.float32)
    o_ref[...] = h * dinv


def _tc2_body(p0_ref, p1_ref, g_ref, d0_ref, d1_ref, b_ref, w_ref, o_ref):
    dinv = lax.rsqrt(d0_ref[0] + d1_ref[0] + 1.0)
    pre = (p0_ref[...] + p1_ref[...] - g_ref[...]) * dinv + b_ref[...]
    z = jnp.maximum(pre, 0.0)
    h = jnp.dot(z, w_ref[...], preferred_element_type=jnp.float32)
    o_ref[...] = h * dinv


def _tc3_body(p0_ref, p1_ref, g_ref, d0_ref, d1_ref, b_ref, o_ref):
    dinv = lax.rsqrt(d0_ref[0] + d1_ref[0] + 1.0)
    o_ref[...] = (p0_ref[...] + p1_ref[...] - g_ref[...]) * dinv + b_ref[...]


_row_spec = pl.BlockSpec((RB, D), lambda i: (i, 0))
_row2_spec = pl.BlockSpec((RB, D), lambda i: (i + GRID, 0))
_d0_spec = pl.BlockSpec((1, RB, 1), lambda i: (0, i, 0))
_d1_spec = pl.BlockSpec((1, RB, 1), lambda i: (1, i, 0))
_w_spec = pl.BlockSpec((D, D), lambda i: (0, 0))
_b_spec = pl.BlockSpec((1, D), lambda i: (0, 0))
_out_t = jax.ShapeDtypeStruct((NP, D), jnp.float32)

_tc1 = pl.pallas_call(
    _tc1_body,
    grid=(GRID,),
    in_specs=[_row_spec, _w_spec, _d0_spec, _d1_spec],
    out_specs=_row_spec,
    out_shape=_out_t,
)

_tc2 = pl.pallas_call(
    _tc2_body,
    grid=(GRID,),
    in_specs=[_row_spec, _row2_spec, _row_spec, _d0_spec, _d1_spec, _b_spec, _w_spec],
    out_specs=_row_spec,
    out_shape=_out_t,
)

_tc3 = pl.pallas_call(
    _tc3_body,
    grid=(GRID,),
    in_specs=[_row_spec, _row2_spec, _row_spec, _d0_spec, _d1_spec, _b_spec],
    out_specs=_row_spec,
    out_shape=_out_t,
)


def kernel(x, edge_index, W1, b1, W2, b2):
    src = edge_index[0].astype(jnp.int32)
    dst = edge_index[1].astype(jnp.int32)
    npad = ER * B - N_EDGES
    pad = jnp.full((npad,), PAD_IDX, jnp.int32)
    src_p = jnp.concatenate([src, pad]).reshape(ER, B)
    dst_p = jnp.concatenate([dst, pad]).reshape(ER, B)
    x_pad = jnp.pad(x, ((0, NP - N_NODES), (0, 0)))
    deg3 = _deg_call(dst_p).reshape(NC, NP, 1)
    g1 = _tc1(x_pad, W1, deg3, deg3)
    p1 = _agg_call(src_p, dst_p, g1)
    g2 = _tc2(p1, p1, g1, deg3, deg3, b1.reshape(1, D), W2)
    p2 = _agg_call(src_p, dst_p, g2)
    out = _tc3(p2, p2, g2, deg3, deg3, b2.reshape(1, D))
    return out[:N_NODES]


# speedup vs baseline: 3.6127x; 3.6127x over previous
import jax
import jax.numpy as jnp
from jax import lax
from jax.experimental import pallas as pl
from jax.experimental.pallas import tpu as pltpu
from jax.experimental.pallas import tpu_sc as plsc

N_NODES = 10000
N_EDGES = 320000
D = 128
NP = 10240
NC, NS = 2, 16
NW = NC * NS
EPT = N_EDGES // NW
B = 80
NCH = EPT // B
MAINC = 120
EPAD = 4 * B
BH = 128
CPTH = 80
ERH = NW * CPTH
PAD_IDX = 10200
RPS = NP // NS
RB = 1024
GRID = NP // RB

_mesh = plsc.VectorSubcoreMesh(
    core_axis_name="c", subcore_axis_name="s", num_cores=NC, num_subcores=NS
)


def _deg_body(dst_hbm, deg_out, dstv, ones_v, zrow_v, acc, sem):
    cid = lax.axis_index("c")
    sid = lax.axis_index("s")
    wid = sid * NC + cid
    one16 = jnp.full((16,), 1.0, jnp.float32)
    zero16 = jnp.zeros((16,), jnp.float32)
    for i in range(BH // 16):
        ones_v[pl.ds(i * 16, 16)] = one16
    for i in range(RPS // 16):
        zrow_v[pl.ds(i * 16, 16)] = zero16
    ro = pl.multiple_of(sid * RPS, 8)
    pltpu.sync_copy(zrow_v, acc.at[pl.ds(ro, RPS)])
    pltpu.sync_copy(dst_hbm.at[pl.ds(wid * CPTH, CPTH)], dstv)
    plsc.subcore_barrier()
    for g in range(CPTH // 16):
        descs = [
            pltpu.async_copy(ones_v, acc.at[dstv.at[g * 16 + i]], sem, add=True)
            for i in range(16)
        ]
        for d in descs:
            d.wait()
    plsc.subcore_barrier()
    wo = pl.multiple_of(cid * NP + sid * RPS, 8)
    pltpu.sync_copy(acc.at[pl.ds(ro, RPS)], deg_out.at[pl.ds(wo, RPS)])


_deg_call = pl.kernel(
    _deg_body,
    out_type=jax.ShapeDtypeStruct((NC * NP,), jnp.float32),
    mesh=_mesh,
    scratch_types=[
        pltpu.VMEM((CPTH, BH), jnp.int32),
        pltpu.VMEM((BH,), jnp.float32),
        pltpu.VMEM((RPS,), jnp.float32),
        pltpu.VMEM_SHARED((NP,), jnp.float32),
        pltpu.SemaphoreType.DMA,
    ],
)


def _agg_body(src_hbm, dst_hbm, g_hbm, out_hbm,
              si0, si1, si2, si3, di0, di1, di2, di3, r0, r1, acc,
              gsem0, gsem1, isem0, isem1, isem2, isem3):
    cid = lax.axis_index("c")
    sid = lax.axis_index("s")
    wid = sid * NC + cid
    sib = (si0, si1, si2, si3)
    dib = (di0, di1, di2, di3)
    rows = (r0, r1)
    gsems = (gsem0, gsem1)
    isems = (isem0, isem1, isem2, isem3)
    ro = pl.multiple_of(sid * RPS, 8)
    pltpu.sync_copy(g_hbm.at[pl.ds(ro, RPS)], acc.at[pl.ds(ro, RPS)])
    plsc.subcore_barrier()
    base = wid * EPT

    def fire_idx(c, p):
        off = pl.multiple_of(base + c * B, 8)
        pltpu.async_copy(src_hbm.at[pl.ds(off, B)], sib[p], isems[p])
        pltpu.async_copy(dst_hbm.at[pl.ds(off, B)], dib[p], isems[p])

    def drain_idx(p):
        pltpu.make_async_copy(src_hbm.at[pl.ds(0, B)], sib[p], isems[p]).wait()
        pltpu.make_async_copy(src_hbm.at[pl.ds(0, B)], dib[p], isems[p]).wait()

    def fire_gather(p4, p2):
        pltpu.async_copy(g_hbm.at[sib[p4]], rows[p2], gsems[p2])

    def drain_gather(p2):
        pltpu.make_async_copy(
            g_hbm.at[pl.ds(0, B)], rows[p2], gsems[p2]
        ).wait()

    def chunk(c, u, fire_idx_ahead, fire_gather_ahead):
        drain_gather(u % 2)
        pltpu.sync_copy(rows[u % 2], acc.at[dib[u]], add=True)
        if fire_idx_ahead:
            fire_idx(c + 4, u)
        if fire_gather_ahead:
            drain_idx((u + 2) % 4)
            fire_gather((u + 2) % 4, u % 2)

    fire_idx(2, 2)
    fire_idx(3, 3)
    o0 = pl.multiple_of(base, 8)
    pltpu.sync_copy(src_hbm.at[pl.ds(o0, B)], si0)
    pltpu.sync_copy(dst_hbm.at[pl.ds(o0, B)], di0)
    pltpu.sync_copy(src_hbm.at[pl.ds(o0 + B, B)], si1)
    pltpu.sync_copy(dst_hbm.at[pl.ds(o0 + B, B)], di1)
    fire_gather(0, 0)
    fire_gather(1, 1)

    def body(k, carry):
        for u in range(4):
            chunk(4 * k + u, u, True, True)
        return carry

    lax.fori_loop(0, MAINC // 4, body, 0)
    for c in range(MAINC, NCH):
        chunk(c, c % 4, c + 4 < NCH, c + 2 < NCH)
    plsc.subcore_barrier()
    wo = pl.multiple_of(cid * NP + sid * RPS, 8)
    pltpu.sync_copy(acc.at[pl.ds(ro, RPS)], out_hbm.at[pl.ds(wo, RPS)])


_agg_call = pl.kernel(
    _agg_body,
    out_type=jax.ShapeDtypeStruct((NC * NP, D), jnp.float32),
    mesh=_mesh,
    scratch_types=[
        pltpu.VMEM((B,), jnp.int32),
        pltpu.VMEM((B,), jnp.int32),
        pltpu.VMEM((B,), jnp.int32),
        pltpu.VMEM((B,), jnp.int32),
        pltpu.VMEM((B,), jnp.int32),
        pltpu.VMEM((B,), jnp.int32),
        pltpu.VMEM((B,), jnp.int32),
        pltpu.VMEM((B,), jnp.int32),
        pltpu.VMEM((B, D), jnp.float32),
        pltpu.VMEM((B, D), jnp.float32),
        pltpu.VMEM_SHARED((NP, D), jnp.float32),
        pltpu.SemaphoreType.DMA,
        pltpu.SemaphoreType.DMA,
        pltpu.SemaphoreType.DMA,
        pltpu.SemaphoreType.DMA,
        pltpu.SemaphoreType.DMA,
        pltpu.SemaphoreType.DMA,
    ],
)


def _tc1_body(x_ref, w_ref, d0_ref, d1_ref, o_ref):
    dinv = lax.rsqrt(d0_ref[0] + d1_ref[0] + 1.0)
    h = jnp.dot(x_ref[...], w_ref[...], preferred_element_type=jnp.float32)
    o_ref[...] = h * dinv


def _tc2_body(p0_ref, p1_ref, g_ref, d0_ref, d1_ref, b_ref, w_ref, o_ref):
    dinv = lax.rsqrt(d0_ref[0] + d1_ref[0] + 1.0)
    pre = (p0_ref[...] + p1_ref[...] - g_ref[...]) * dinv + b_ref[...]
    z = jnp.maximum(pre, 0.0)
    h = jnp.dot(z, w_ref[...], preferred_element_type=jnp.float32)
    o_ref[...] = h * dinv


def _tc3_body(p0_ref, p1_ref, g_ref, d0_ref, d1_ref, b_ref, o_ref):
    dinv = lax.rsqrt(d0_ref[0] + d1_ref[0] + 1.0)
    o_ref[...] = (p0_ref[...] + p1_ref[...] - g_ref[...]) * dinv + b_ref[...]


_row_spec = pl.BlockSpec((RB, D), lambda i: (i, 0))
_row2_spec = pl.BlockSpec((RB, D), lambda i: (i + GRID, 0))
_d0_spec = pl.BlockSpec((1, RB, 1), lambda i: (0, i, 0))
_d1_spec = pl.BlockSpec((1, RB, 1), lambda i: (1, i, 0))
_w_spec = pl.BlockSpec((D, D), lambda i: (0, 0))
_b_spec = pl.BlockSpec((1, D), lambda i: (0, 0))
_out_t = jax.ShapeDtypeStruct((NP, D), jnp.float32)

_tc1 = pl.pallas_call(
    _tc1_body,
    grid=(GRID,),
    in_specs=[_row_spec, _w_spec, _d0_spec, _d1_spec],
    out_specs=_row_spec,
    out_shape=_out_t,
)

_tc2 = pl.pallas_call(
    _tc2_body,
    grid=(GRID,),
    in_specs=[_row_spec, _row2_spec, _row_spec, _d0_spec, _d1_spec, _b_spec, _w_spec],
    out_specs=_row_spec,
    out_shape=_out_t,
)

_tc3 = pl.pallas_call(
    _tc3_body,
    grid=(GRID,),
    in_specs=[_row_spec, _row2_spec, _row_spec, _d0_spec, _d1_spec, _b_spec],
    out_specs=_row_spec,
    out_shape=_out_t,
)


def kernel(x, edge_index, W1, b1, W2, b2):
    src = edge_index[0].astype(jnp.int32)
    dst = edge_index[1].astype(jnp.int32)
    zpad = jnp.zeros((EPAD,), jnp.int32)
    src_f = jnp.concatenate([src, zpad])
    dst_f = jnp.concatenate([dst, zpad])
    hpad = jnp.full((ERH * BH - N_EDGES,), PAD_IDX, jnp.int32)
    dst_h = jnp.concatenate([dst, hpad]).reshape(ERH, BH)
    x_pad = jnp.pad(x, ((0, NP - N_NODES), (0, 0)))
    deg3 = _deg_call(dst_h).reshape(NC, NP, 1)
    g1 = _tc1(x_pad, W1, deg3, deg3)
    p1 = _agg_call(src_f, dst_f, g1)
    g2 = _tc2(p1, p1, g1, deg3, deg3, b1.reshape(1, D), W2)
    p2 = _agg_call(src_f, dst_f, g2)
    out = _tc3(p2, p2, g2, deg3, deg3, b2.reshape(1, D))
    return out[:N_NODES]


# speedup vs baseline: 3.6927x; 1.0221x over previous
import jax
import jax.numpy as jnp
from jax import lax
from jax.experimental import pallas as pl
from jax.experimental.pallas import tpu as pltpu
from jax.experimental.pallas import tpu_sc as plsc

N_NODES = 10000
N_EDGES = 320000
D = 128
NP = 10240
NC, NS = 2, 16
NW = NC * NS
EPT = N_EDGES // NW
B = 80
NCH = EPT // B
MAINC = 120
EPAD = 4 * B
BH = 128
CPTH = 80
ERH = NW * CPTH
PAD_IDX = 10200
RPS = NP // NS
RB = 1024
GRID = NP // RB

_mesh = plsc.VectorSubcoreMesh(
    core_axis_name="c", subcore_axis_name="s", num_cores=NC, num_subcores=NS
)


def _deg_body(dst_hbm, deg_out, dstv, ones_v, zrow_v, acc, sem):
    cid = lax.axis_index("c")
    sid = lax.axis_index("s")
    wid = sid * NC + cid
    one16 = jnp.full((16,), 1.0, jnp.float32)
    zero16 = jnp.zeros((16,), jnp.float32)
    for i in range(BH // 16):
        ones_v[pl.ds(i * 16, 16)] = one16
    for i in range(RPS // 16):
        zrow_v[pl.ds(i * 16, 16)] = zero16
    ro = pl.multiple_of(sid * RPS, 8)
    pltpu.sync_copy(zrow_v, acc.at[pl.ds(ro, RPS)])
    pltpu.sync_copy(dst_hbm.at[pl.ds(wid * CPTH, CPTH)], dstv)
    plsc.subcore_barrier()
    for g in range(CPTH // 16):
        descs = [
            pltpu.async_copy(ones_v, acc.at[dstv.at[g * 16 + i]], sem, add=True)
            for i in range(16)
        ]
        for d in descs:
            d.wait()
    plsc.subcore_barrier()
    wo = pl.multiple_of(cid * NP + sid * RPS, 8)
    pltpu.sync_copy(acc.at[pl.ds(ro, RPS)], deg_out.at[pl.ds(wo, RPS)])


_deg_call = pl.kernel(
    _deg_body,
    out_type=jax.ShapeDtypeStruct((NC * NP,), jnp.float32),
    mesh=_mesh,
    scratch_types=[
        pltpu.VMEM((CPTH, BH), jnp.int32),
        pltpu.VMEM((BH,), jnp.float32),
        pltpu.VMEM((RPS,), jnp.float32),
        pltpu.VMEM_SHARED((NP,), jnp.float32),
        pltpu.SemaphoreType.DMA,
    ],
)


def _agg_body(src_hbm, dst_hbm, g_hbm, out_hbm, sivs, divs, rowvs, acc,
              gsems, ssems, isems):
    cid = lax.axis_index("c")
    sid = lax.axis_index("s")
    wid = sid * NC + cid
    ro = pl.multiple_of(sid * RPS, 8)
    pltpu.sync_copy(g_hbm.at[pl.ds(ro, RPS)], acc.at[pl.ds(ro, RPS)])
    plsc.subcore_barrier()
    base = wid * EPT

    def fire_idx(c, s8):
        off = pl.multiple_of(base + c * B, 8)
        pltpu.async_copy(src_hbm.at[pl.ds(off, B)], sivs[s8], isems[s8])
        pltpu.async_copy(dst_hbm.at[pl.ds(off, B)], divs[s8], isems[s8])

    def drain_idx(s8):
        pltpu.make_async_copy(src_hbm.at[pl.ds(0, B)], sivs[s8], isems[s8]).wait()
        pltpu.make_async_copy(src_hbm.at[pl.ds(0, B)], divs[s8], isems[s8]).wait()

    def fire_gather(s8, r4):
        pltpu.async_copy(g_hbm.at[sivs[s8]], rowvs[r4], gsems[r4])

    def drain_gather(r4):
        pltpu.make_async_copy(g_hbm.at[pl.ds(0, B)], rowvs[r4], gsems[r4]).wait()

    def drain_scatter(r4):
        pltpu.make_async_copy(g_hbm.at[pl.ds(0, B)], rowvs[r4], ssems[r4]).wait()

    def chunk(c, u, do_s4, do_i5, do_g7):
        r = u % 4
        drain_gather(r)
        pltpu.async_copy(rowvs[r], acc.at[divs[u]], ssems[r], add=True)
        if do_s4:
            drain_scatter((r + 2) % 4)
        if do_i5:
            fire_idx(c + 6, (u + 6) % 8)
        if do_g7:
            drain_idx((u + 2) % 8)
            fire_gather((u + 2) % 8, (r + 2) % 4)

    for c in range(2, 8):
        fire_idx(c, c)
    o0 = pl.multiple_of(base, 8)
    pltpu.sync_copy(src_hbm.at[pl.ds(o0, B)], sivs[0])
    pltpu.sync_copy(dst_hbm.at[pl.ds(o0, B)], divs[0])
    pltpu.sync_copy(src_hbm.at[pl.ds(o0 + B, B)], sivs[1])
    pltpu.sync_copy(dst_hbm.at[pl.ds(o0 + B, B)], divs[1])
    fire_gather(0, 0)
    fire_gather(1, 1)
    for c in range(8):
        chunk(c, c, c >= 2, c >= 2, True)

    def body(k, carry):
        for u in range(8):
            chunk(8 * k + u, u, True, True, True)
        return carry

    lax.fori_loop(1, MAINC // 8, body, 0)
    for c in range(MAINC, NCH):
        chunk(c, c % 8, True, False, c + 2 < NCH)
    drain_scatter(3)
    drain_scatter(0)
    drain_idx(5)
    plsc.subcore_barrier()
    wo = pl.multiple_of(cid * NP + sid * RPS, 8)
    pltpu.sync_copy(acc.at[pl.ds(ro, RPS)], out_hbm.at[pl.ds(wo, RPS)])


_agg_call = pl.kernel(
    _agg_body,
    out_type=jax.ShapeDtypeStruct((NC * NP, D), jnp.float32),
    mesh=_mesh,
    scratch_types=[
        [pltpu.VMEM((B,), jnp.int32)] * 8,
        [pltpu.VMEM((B,), jnp.int32)] * 8,
        [pltpu.VMEM((B, D), jnp.float32)] * 4,
        pltpu.VMEM_SHARED((NP, D), jnp.float32),
        [pltpu.SemaphoreType.DMA] * 4,
        [pltpu.SemaphoreType.DMA] * 4,
        [pltpu.SemaphoreType.DMA] * 8,
    ],
)


def _tc1_body(x_ref, w_ref, d0_ref, d1_ref, o_ref):
    dinv = lax.rsqrt(d0_ref[0] + d1_ref[0] + 1.0)
    h = jnp.dot(x_ref[...], w_ref[...], preferred_element_type=jnp.float32)
    o_ref[...] = h * dinv


def _tc2_body(p0_ref, p1_ref, g_ref, d0_ref, d1_ref, b_ref, w_ref, o_ref):
    dinv = lax.rsqrt(d0_ref[0] + d1_ref[0] + 1.0)
    pre = (p0_ref[...] + p1_ref[...] - g_ref[...]) * dinv + b_ref[...]
    z = jnp.maximum(pre, 0.0)
    h = jnp.dot(z, w_ref[...], preferred_element_type=jnp.float32)
    o_ref[...] = h * dinv


def _tc3_body(p0_ref, p1_ref, g_ref, d0_ref, d1_ref, b_ref, o_ref):
    dinv = lax.rsqrt(d0_ref[0] + d1_ref[0] + 1.0)
    o_ref[...] = (p0_ref[...] + p1_ref[...] - g_ref[...]) * dinv + b_ref[...]


_row_spec = pl.BlockSpec((RB, D), lambda i: (i, 0))
_row2_spec = pl.BlockSpec((RB, D), lambda i: (i + GRID, 0))
_d0_spec = pl.BlockSpec((1, RB, 1), lambda i: (0, i, 0))
_d1_spec = pl.BlockSpec((1, RB, 1), lambda i: (1, i, 0))
_w_spec = pl.BlockSpec((D, D), lambda i: (0, 0))
_b_spec = pl.BlockSpec((1, D), lambda i: (0, 0))
_out_t = jax.ShapeDtypeStruct((NP, D), jnp.float32)

_tc1 = pl.pallas_call(
    _tc1_body,
    grid=(GRID,),
    in_specs=[_row_spec, _w_spec, _d0_spec, _d1_spec],
    out_specs=_row_spec,
    out_shape=_out_t,
)

_tc2 = pl.pallas_call(
    _tc2_body,
    grid=(GRID,),
    in_specs=[_row_spec, _row2_spec, _row_spec, _d0_spec, _d1_spec, _b_spec, _w_spec],
    out_specs=_row_spec,
    out_shape=_out_t,
)

_tc3 = pl.pallas_call(
    _tc3_body,
    grid=(GRID,),
    in_specs=[_row_spec, _row2_spec, _row_spec, _d0_spec, _d1_spec, _b_spec],
    out_specs=_row_spec,
    out_shape=_out_t,
)


def kernel(x, edge_index, W1, b1, W2, b2):
    src = edge_index[0].astype(jnp.int32)
    dst = edge_index[1].astype(jnp.int32)
    zpad = jnp.zeros((EPAD,), jnp.int32)
    src_f = jnp.concatenate([src, zpad])
    dst_f = jnp.concatenate([dst, zpad])
    hpad = jnp.full((ERH * BH - N_EDGES,), PAD_IDX, jnp.int32)
    dst_h = jnp.concatenate([dst, hpad]).reshape(ERH, BH)
    x_pad = jnp.pad(x, ((0, NP - N_NODES), (0, 0)))
    deg3 = _deg_call(dst_h).reshape(NC, NP, 1)
    g1 = _tc1(x_pad, W1, deg3, deg3)
    p1 = _agg_call(src_f, dst_f, g1)
    g2 = _tc2(p1, p1, g1, deg3, deg3, b1.reshape(1, D), W2)
    p2 = _agg_call(src_f, dst_f, g2)
    out = _tc3(p2, p2, g2, deg3, deg3, b2.reshape(1, D))
    return out[:N_NODES]
